# baseline (speedup 1.0000x reference)
import functools

import jax
import jax.numpy as jnp
from jax.experimental import pallas as pl
from jax.experimental.pallas import tpu as pltpu

B, H, W = 8, 128, 128
C, K, D = 19, 1, 512
Q = B * H * W
BQ = 8192
BH = BQ // W
PB = H // BH
HALVES = 2
BHALF = B // HALVES
QHALF = Q // HALVES
STEPS_HALF = QHALF // BQ


def _fused_body(emb_ref, logits_ref, means_ref, npw_ref,
                nm_ref, score_ref, py_ref, ps_ref):
    e = emb_ref[...]
    m = means_ref[:, 0, :]
    lgT = logits_ref[...]
    w = npw_ref[...]

    crossT = jax.lax.dot_general(
        m, e, (((1,), (1,)), ((), ())),
        preferred_element_type=jnp.float32)
    ee = e * e
    q2T = jax.lax.dot_general(
        jnp.ones((1, D), jnp.float32), ee, (((1,), (1,)), ((), ())),
        preferred_element_type=jnp.float32)
    m2 = jnp.sum(m * m, axis=1, keepdims=True)

    nmT = q2T + m2 - 2.0 * crossT
    simT = 1.0 / (1.0 + 0.5 * nmT)
    w0 = w[:, 0:1]
    w1 = w[:, 1:2]
    w2 = w[:, 2:3]
    scoreT = jax.nn.sigmoid(w0 * lgT + w1 * simT + w2)

    mxT = jnp.max(lgT, axis=0, keepdims=True)
    iotaT = jax.lax.broadcasted_iota(jnp.int32, lgT.shape, 0)
    pyT = jnp.min(jnp.where(lgT == mxT, iotaT, C), axis=0,
                  keepdims=True)
    onehotT = iotaT == pyT
    psT = jnp.sum(jnp.where(onehotT, scoreT, 0.0), axis=0,
                  keepdims=True)

    nm_ref[...] = nmT
    score_ref[...] = scoreT
    py_ref[...] = pyT.astype(jnp.float32).reshape(1, BH, W)
    ps_ref[...] = psT.reshape(1, BH, W)


def _half_call(emb, logitsT, means, npw_pad, half):
    base = half * STEPS_HALF
    return pl.pallas_call(
        _fused_body,
        grid=(STEPS_HALF,),
        in_specs=[
            pl.BlockSpec((BQ, D), lambda i: (base + i, 0)),
            pl.BlockSpec((C, BQ), lambda i: (0, base + i)),
            pl.BlockSpec((C, K, D), lambda i: (0, 0, 0)),
            pl.BlockSpec((C, 128), lambda i: (0, 0)),
        ],
        out_specs=[
            pl.BlockSpec((C, BQ), lambda i: (0, i)),
            pl.BlockSpec((C, BQ), lambda i: (0, i)),
            pl.BlockSpec((1, BH, W), lambda i: (i // PB, i % PB, 0)),
            pl.BlockSpec((1, BH, W), lambda i: (i // PB, i % PB, 0)),
        ],
        out_shape=[
            jax.ShapeDtypeStruct((C, QHALF), jnp.float32),
            jax.ShapeDtypeStruct((C, QHALF), jnp.float32),
            jax.ShapeDtypeStruct((BHALF, H, W), jnp.float32),
            jax.ShapeDtypeStruct((BHALF, H, W), jnp.float32),
        ],
        compiler_params=pltpu.CompilerParams(
            dimension_semantics=("arbitrary",)),
    )(emb, logitsT, means, npw_pad)


def kernel(emb, logits, means, np_w):
    logitsT = logits.T
    npw_pad = jnp.pad(np_w, ((0, 0), (0, 128 - 3)))

    pys, pss, scores, nms = [], [], [], []
    for half in range(HALVES):
        nmT, scoreT, py, ps = _half_call(emb, logitsT, means, npw_pad, half)
        nms.append(nmT.T.reshape(BHALF, H, W, C))
        scores.append(scoreT.T.reshape(BHALF, H, W, C))
        pys.append(py)
        pss.append(ps)

    pred_y_f = jnp.concatenate(pys, axis=0)
    pred_score_r = jnp.concatenate(pss, axis=0)
    pred_score_all = jnp.concatenate(scores, axis=0)
    nm_dist_r = jnp.concatenate(nms, axis=0)
    logits_r = logits.reshape(B, H, W, C)
    return (pred_y_f, pred_score_r, pred_score_all, nm_dist_r, logits_r)


# speedup vs baseline: 1.0856x; 1.0856x over previous
import jax
import jax.numpy as jnp
from jax.experimental import pallas as pl
from jax.experimental.pallas import tpu as pltpu

B, H, W = 8, 128, 128
C, K, D = 19, 1, 512
Q = B * H * W
BQ = 8192
BH = BQ // W
PB = H // BH


def _fused_body(emb_ref, logits_ref, means_ref, npw_ref,
                nm_ref, score_ref, py_ref, ps_ref):
    e = emb_ref[...]
    m = means_ref[:, 0, :]
    lgT = logits_ref[...]
    w = npw_ref[...]

    crossT = jax.lax.dot_general(
        m, e, (((1,), (1,)), ((), ())),
        preferred_element_type=jnp.float32)
    ee = e * e
    q2T = jax.lax.dot_general(
        jnp.ones((1, D), jnp.float32), ee, (((1,), (1,)), ((), ())),
        preferred_element_type=jnp.float32)
    m2 = jnp.sum(m * m, axis=1, keepdims=True)

    nmT = q2T + m2 - 2.0 * crossT
    simT = 1.0 / (1.0 + 0.5 * nmT)
    w0 = w[:, 0:1]
    w1 = w[:, 1:2]
    w2 = w[:, 2:3]
    scoreT = jax.nn.sigmoid(w0 * lgT + w1 * simT + w2)

    mxT = jnp.max(lgT, axis=0, keepdims=True)
    iotaT = jax.lax.broadcasted_iota(jnp.int32, lgT.shape, 0)
    pyT = jnp.min(jnp.where(lgT == mxT, iotaT, C), axis=0,
                  keepdims=True)
    onehotT = iotaT == pyT
    psT = jnp.sum(jnp.where(onehotT, scoreT, 0.0), axis=0,
                  keepdims=True)

    nm_ref[...] = nmT
    score_ref[...] = scoreT
    py_ref[...] = pyT.astype(jnp.float32).reshape(1, BH, W)
    ps_ref[...] = psT.reshape(1, BH, W)


def kernel(emb, logits, means, np_w):
    logitsT = logits.T
    npw_pad = jnp.pad(np_w, ((0, 0), (0, 128 - 3)))
    grid = (Q // BQ,)
    nmT, scoreT, py, ps = pl.pallas_call(
        _fused_body,
        grid=grid,
        in_specs=[
            pl.BlockSpec((BQ, D), lambda i: (i, 0)),
            pl.BlockSpec((C, BQ), lambda i: (0, i)),
            pl.BlockSpec((C, K, D), lambda i: (0, 0, 0)),
            pl.BlockSpec((C, 128), lambda i: (0, 0)),
        ],
        out_specs=[
            pl.BlockSpec((C, BQ), lambda i: (0, i)),
            pl.BlockSpec((C, BQ), lambda i: (0, i)),
            pl.BlockSpec((1, BH, W), lambda i: (i // PB, i % PB, 0)),
            pl.BlockSpec((1, BH, W), lambda i: (i // PB, i % PB, 0)),
        ],
        out_shape=[
            jax.ShapeDtypeStruct((C, Q), jnp.float32),
            jax.ShapeDtypeStruct((C, Q), jnp.float32),
            jax.ShapeDtypeStruct((B, H, W), jnp.float32),
            jax.ShapeDtypeStruct((B, H, W), jnp.float32),
        ],
        compiler_params=pltpu.CompilerParams(
            dimension_semantics=("parallel",)),
    )(emb, logitsT, means, npw_pad)

    pred_score_all = scoreT.T.reshape(B, H, W, C)
    nm_dist_r = nmT.T.reshape(B, H, W, C)
    logits_r = logits.reshape(B, H, W, C)
    return (py, ps, pred_score_all, nm_dist_r, logits_r)
